# baseline (speedup 1.0000x reference)
import functools

import jax
import jax.numpy as jnp
import numpy as np
from jax import lax
from jax.experimental import pallas as pl
from jax.experimental.pallas import tpu as pltpu
from jax.experimental.pallas import tpu_sc as plsc

DELTA_VAR = 1.0
DELTA_DIST = 2.0


def _sc_segsum(D, H, W, K, NC, NW, data_hbm, lab_hbm, sums_pp, counts_pp,
               dbuf, lbuf, acc, acc2, cred):
    HPW = H // NW
    HPB = 2
    NBLK = HPW // HPB
    GRP = (HPB * W) // 16

    wid = lax.axis_index("s") * NC + lax.axis_index("c")
    lane = lax.iota(jnp.int32, 16)
    ones = jnp.full((16,), 1.0, jnp.float32)
    zeros = jnp.zeros((16,), jnp.float32)

    def zero_body(j, c):
        acc[pl.ds(j * 16, 16)] = zeros
        return c
    lax.fori_loop(0, (K * D * 16) // 16, zero_body, 0)
    for j in range(K):
        acc[pl.ds(K * D * 16 + j * 16, 16)] = zeros

    h_base = wid * HPW
    for blk in range(NBLK):
        pltpu.sync_copy(data_hbm.at[:, pl.ds(h_base + blk * HPB, HPB), :],
                        dbuf)
        pltpu.sync_copy(lab_hbm.at[pl.ds(h_base + blk * HPB, HPB), :], lbuf)
        for row in range(HPB):
            def g_body(g, c):
                col = g * 16
                lab16 = lbuf[row, pl.ds(col, 16)]
                base = lab16 * 16 + lane
                plsc.addupdate_scatter(acc, [K * D * 16 + base], ones)
                for d in range(D):
                    v = dbuf[d, row, pl.ds(col, 16)]
                    plsc.addupdate_scatter(acc, [base + d * (K * 16)], v)
                return c
            lax.fori_loop(0, W // 16, g_body, 0)

    def red_body(dd, c):
        res = zeros
        for labk in range(K):
            row16 = acc[pl.ds(dd * (K * 16) + labk * 16, 16)]
            s = jnp.sum(row16)
            res = jnp.where(lane == labk, s, res)
        acc2[dd, :] = res
        return c
    lax.fori_loop(0, D, red_body, 0)
    cres = zeros
    for labk in range(K):
        s = jnp.sum(acc[pl.ds(K * D * 16 + labk * 16, 16)])
        cres = jnp.where(lane == labk, s, cres)
    cred[...] = cres

    pltpu.sync_copy(acc2, sums_pp.at[wid])
    pltpu.sync_copy(cred, counts_pp.at[wid])


def _phase2_body(K, NW, NBLK, data_ref, lab_ref, sums_ref, counts_ref,
                 out_ref):
    i = pl.program_id(0)
    hb, w = lab_ref.shape
    D = sums_ref.shape[1]
    sums = jnp.sum(sums_ref[...], axis=0)
    counts = jnp.sum(counts_ref[...], axis=0, keepdims=True)
    centers = sums / counts
    x = data_ref[...].reshape(D, hb * w)
    lab2 = lab_ref[...].reshape(1, hb * w)
    onehot = (jax.lax.broadcasted_iota(jnp.int32, (K, hb * w), 0)
              == lab2).astype(jnp.float32)
    c_sel = jax.lax.dot_general(
        centers, onehot, (((1,), (0,)), ((), ())),
        preferred_element_type=jnp.float32)
    diff = x - c_sel
    norm2 = jnp.sum(diff * diff, axis=0, keepdims=True)
    norm = jnp.sqrt(norm2)
    h = jnp.maximum(norm - DELTA_VAR, 0.0)
    var_b = jnp.sum(h * h)

    @pl.when(i == 0)
    def _():
        out_ref[0, 0] = 0.0

    out_ref[0, 0] += var_b / K

    @pl.when(i == NBLK - 1)
    def _():
        delta_reg = float(np.sqrt(centers.shape[0]))
        n2 = jnp.sum(centers * centers, axis=0)
        gram = jax.lax.dot_general(
            centers, centers, (((0,), (0,)), ((), ())),
            preferred_element_type=jnp.float32)
        sq = jnp.maximum(n2[:, None] + n2[None, :] - 2.0 * gram, 0.0)
        eye = jnp.eye(K, dtype=jnp.float32)
        cnorm = jnp.sqrt(sq + eye)
        hinge = (jnp.maximum(2.0 * DELTA_DIST - cnorm, 0.0) ** 2) * (1.0 - eye)
        dist_term = jnp.sum(hinge) / (K * (K - 1))
        reg_term = jnp.sum(jnp.maximum(jnp.sqrt(n2) - delta_reg, 0.0)) / K
        out_ref[0, 0] += dist_term + reg_term


def kernel(data, labels, cluster_ids):
    D, H, W = data.shape
    K = cluster_ids.shape[0]
    NC, NS = 2, 16
    NW = NC * NS
    HPB = 2

    mesh = plsc.VectorSubcoreMesh(core_axis_name="c", subcore_axis_name="s")
    sc_fn = pl.kernel(
        functools.partial(_sc_segsum, D, H, W, K, NC, NW),
        mesh=mesh,
        compiler_params=pltpu.CompilerParams(needs_layout_passes=False),
        out_type=[
            jax.ShapeDtypeStruct((NW, D, K), jnp.float32),
            jax.ShapeDtypeStruct((NW, K), jnp.float32),
        ],
        scratch_types=[
            pltpu.VMEM((D, HPB, W), jnp.float32),
            pltpu.VMEM((HPB, W), jnp.int32),
            pltpu.VMEM((K * D * 16 + K * 16,), jnp.float32),
            pltpu.VMEM((D, K), jnp.float32),
            pltpu.VMEM((K,), jnp.float32),
        ],
    )
    sums_pp, counts_pp = sc_fn(data, labels)

    HB = 16
    NBLK = H // HB
    out = pl.pallas_call(
        functools.partial(_phase2_body, K, NW, NBLK),
        grid=(NBLK,),
        in_specs=[
            pl.BlockSpec((D, HB, W), lambda i: (0, i, 0)),
            pl.BlockSpec((HB, W), lambda i: (i, 0)),
            pl.BlockSpec((NW, D, K), lambda i: (0, 0, 0)),
            pl.BlockSpec((NW, K), lambda i: (0, 0)),
        ],
        out_specs=pl.BlockSpec(memory_space=pltpu.SMEM),
        out_shape=jax.ShapeDtypeStruct((1, 1), jnp.float32),
    )(data, labels, sums_pp, counts_pp)

    return out[0, 0]


# speedup vs baseline: 1.4942x; 1.4942x over previous
import functools

import jax
import jax.numpy as jnp
import numpy as np
from jax import lax
from jax.experimental import pallas as pl
from jax.experimental.pallas import tpu as pltpu
from jax.experimental.pallas import tpu_sc as plsc

DELTA_VAR = 1.0
DELTA_DIST = 2.0


def _sc_segsum(D, H, W, K, NC, NW, data_hbm, lab_hbm, sums_pp, counts_pp,
               dbuf, lbuf, acc, acc2, cred):
    HPW = H // NW
    HPB = 2
    NBLK = HPW // HPB
    GRP = (HPB * W) // 16

    wid = lax.axis_index("s") * NC + lax.axis_index("c")
    lane = lax.iota(jnp.int32, 16)
    ones = jnp.full((16,), 1.0, jnp.float32)
    zeros = jnp.zeros((16,), jnp.float32)

    def zero_body(j, c):
        acc[pl.ds(j * 16, 16)] = zeros
        return c
    lax.fori_loop(0, (K * D * 16) // 16, zero_body, 0)
    for j in range(K):
        acc[pl.ds(K * D * 16 + j * 16, 16)] = zeros

    h_base = wid * HPW
    for blk in range(NBLK):
        pltpu.sync_copy(data_hbm.at[:, pl.ds(h_base + blk * HPB, HPB), :],
                        dbuf)
        pltpu.sync_copy(lab_hbm.at[pl.ds(h_base + blk * HPB, HPB), :], lbuf)
        for row in range(HPB):
            @plsc.parallel_loop(0, W // 16)
            def g_body(g):
                col = g * 16
                lab16 = lbuf[row, pl.ds(col, 16)]
                base = lab16 * 16 + lane
                plsc.addupdate_scatter(acc, [K * D * 16 + base], ones)
                for d in range(D):
                    v = dbuf[d, row, pl.ds(col, 16)]
                    plsc.addupdate_scatter(acc, [base + d * (K * 16)], v)

    def red_body(dd, c):
        res = zeros
        for labk in range(K):
            row16 = acc[pl.ds(dd * (K * 16) + labk * 16, 16)]
            s = jnp.sum(row16)
            res = jnp.where(lane == labk, s, res)
        acc2[dd, :] = res
        return c
    lax.fori_loop(0, D, red_body, 0)
    cres = zeros
    for labk in range(K):
        s = jnp.sum(acc[pl.ds(K * D * 16 + labk * 16, 16)])
        cres = jnp.where(lane == labk, s, cres)
    cred[...] = cres

    pltpu.sync_copy(acc2, sums_pp.at[wid])
    pltpu.sync_copy(cred, counts_pp.at[wid])


def _phase2_body(K, NW, NBLK, data_ref, lab_ref, sums_ref, counts_ref,
                 out_ref):
    i = pl.program_id(0)
    hb, w = lab_ref.shape
    D = sums_ref.shape[1]
    sums = jnp.sum(sums_ref[...], axis=0)
    counts = jnp.sum(counts_ref[...], axis=0, keepdims=True)
    centers = sums / counts
    x = data_ref[...].reshape(D, hb * w)
    lab2 = lab_ref[...].reshape(1, hb * w)
    onehot = (jax.lax.broadcasted_iota(jnp.int32, (K, hb * w), 0)
              == lab2).astype(jnp.float32)
    c_sel = jax.lax.dot_general(
        centers, onehot, (((1,), (0,)), ((), ())),
        preferred_element_type=jnp.float32)
    diff = x - c_sel
    norm2 = jnp.sum(diff * diff, axis=0, keepdims=True)
    norm = jnp.sqrt(norm2)
    h = jnp.maximum(norm - DELTA_VAR, 0.0)
    var_b = jnp.sum(h * h)

    @pl.when(i == 0)
    def _():
        out_ref[0, 0] = 0.0

    out_ref[0, 0] += var_b / K

    @pl.when(i == NBLK - 1)
    def _():
        delta_reg = float(np.sqrt(centers.shape[0]))
        n2 = jnp.sum(centers * centers, axis=0)
        gram = jax.lax.dot_general(
            centers, centers, (((0,), (0,)), ((), ())),
            preferred_element_type=jnp.float32)
        sq = jnp.maximum(n2[:, None] + n2[None, :] - 2.0 * gram, 0.0)
        eye = jnp.eye(K, dtype=jnp.float32)
        cnorm = jnp.sqrt(sq + eye)
        hinge = (jnp.maximum(2.0 * DELTA_DIST - cnorm, 0.0) ** 2) * (1.0 - eye)
        dist_term = jnp.sum(hinge) / (K * (K - 1))
        reg_term = jnp.sum(jnp.maximum(jnp.sqrt(n2) - delta_reg, 0.0)) / K
        out_ref[0, 0] += dist_term + reg_term


def kernel(data, labels, cluster_ids):
    D, H, W = data.shape
    K = cluster_ids.shape[0]
    NC, NS = 2, 16
    NW = NC * NS
    HPB = 2

    mesh = plsc.VectorSubcoreMesh(core_axis_name="c", subcore_axis_name="s")
    sc_fn = pl.kernel(
        functools.partial(_sc_segsum, D, H, W, K, NC, NW),
        mesh=mesh,
        compiler_params=pltpu.CompilerParams(needs_layout_passes=False),
        out_type=[
            jax.ShapeDtypeStruct((NW, D, K), jnp.float32),
            jax.ShapeDtypeStruct((NW, K), jnp.float32),
        ],
        scratch_types=[
            pltpu.VMEM((D, HPB, W), jnp.float32),
            pltpu.VMEM((HPB, W), jnp.int32),
            pltpu.VMEM((K * D * 16 + K * 16,), jnp.float32),
            pltpu.VMEM((D, K), jnp.float32),
            pltpu.VMEM((K,), jnp.float32),
        ],
    )
    sums_pp, counts_pp = sc_fn(data, labels)

    HB = 16
    NBLK = H // HB
    out = pl.pallas_call(
        functools.partial(_phase2_body, K, NW, NBLK),
        grid=(NBLK,),
        in_specs=[
            pl.BlockSpec((D, HB, W), lambda i: (0, i, 0)),
            pl.BlockSpec((HB, W), lambda i: (i, 0)),
            pl.BlockSpec((NW, D, K), lambda i: (0, 0, 0)),
            pl.BlockSpec((NW, K), lambda i: (0, 0)),
        ],
        out_specs=pl.BlockSpec(memory_space=pltpu.SMEM),
        out_shape=jax.ShapeDtypeStruct((1, 1), jnp.float32),
    )(data, labels, sums_pp, counts_pp)

    return out[0, 0]


# speedup vs baseline: 1.8348x; 1.2280x over previous
import functools

import jax
import jax.numpy as jnp
import numpy as np
from jax import lax
from jax.experimental import pallas as pl
from jax.experimental.pallas import tpu as pltpu
from jax.experimental.pallas import tpu_sc as plsc

DELTA_VAR = 1.0
DELTA_DIST = 2.0


def _sc_segsum(D, H, W, K, NC, NW, HPB, data_hbm, lab_hbm, sums_pp, counts_pp,
               dbuf0, dbuf1, lbuf0, lbuf1, acc, acc2, cred,
               sem0, sem1, lsem0, lsem1):
    HPW = H // NW
    NBLK = HPW // HPB

    wid = lax.axis_index("s") * NC + lax.axis_index("c")
    lane = lax.iota(jnp.int32, 16)
    ones = jnp.full((16,), 1.0, jnp.float32)
    zeros = jnp.zeros((16,), jnp.float32)

    @plsc.parallel_loop(0, (K * D * 16 + K * 16) // 16)
    def zero_body(j):
        acc[pl.ds(j * 16, 16)] = zeros

    dbufs = (dbuf0, dbuf1)
    lbufs = (lbuf0, lbuf1)
    sems = (sem0, sem1)
    lsems = (lsem0, lsem1)
    h_base = wid * HPW

    def start(blk):
        b = blk % 2
        pltpu.async_copy(data_hbm.at[:, pl.ds(h_base + blk * HPB, HPB), :],
                         dbufs[b], sems[b])
        pltpu.async_copy(lab_hbm.at[pl.ds(h_base + blk * HPB, HPB), :],
                         lbufs[b], lsems[b])

    def wait(blk):
        b = blk % 2
        pltpu.make_async_copy(data_hbm.at[:, pl.ds(h_base + blk * HPB, HPB), :],
                              dbufs[b], sems[b]).wait()
        pltpu.make_async_copy(lab_hbm.at[pl.ds(h_base + blk * HPB, HPB), :],
                              lbufs[b], lsems[b]).wait()

    start(0)
    for blk in range(NBLK):
        b = blk % 2
        wait(blk)
        if blk + 1 < NBLK:
            start(blk + 1)
        dbuf = dbufs[b]
        lbuf = lbufs[b]
        for row in range(HPB):
            @plsc.parallel_loop(0, W // 16)
            def g_body(g):
                col = g * 16
                lab16 = lbuf[row, pl.ds(col, 16)]
                base = lab16 * 16 + lane
                plsc.addupdate_scatter(acc, [K * D * 16 + base], ones)
                for d in range(D):
                    v = dbuf[d, row, pl.ds(col, 16)]
                    plsc.addupdate_scatter(acc, [base + d * (K * 16)], v)

    @plsc.parallel_loop(0, D)
    def red_body(dd):
        res = zeros
        for labk in range(K):
            row16 = acc[pl.ds(dd * (K * 16) + labk * 16, 16)]
            s = jnp.sum(row16)
            res = jnp.where(lane == labk, s, res)
        acc2[dd, :] = res

    cres = zeros
    for labk in range(K):
        s = jnp.sum(acc[pl.ds(K * D * 16 + labk * 16, 16)])
        cres = jnp.where(lane == labk, s, cres)
    cred[...] = cres

    pltpu.sync_copy(acc2, sums_pp.at[wid])
    pltpu.sync_copy(cred, counts_pp.at[wid])


def _phase2_body(K, NW, NBLK, data_ref, lab_ref, sums_ref, counts_ref,
                 out_ref):
    i = pl.program_id(0)
    hb, w = lab_ref.shape
    D = sums_ref.shape[1]
    sums = jnp.sum(sums_ref[...], axis=0)
    counts = jnp.sum(counts_ref[...], axis=0, keepdims=True)
    centers = sums / counts
    x = data_ref[...].reshape(D, hb * w)
    lab2 = lab_ref[...].reshape(1, hb * w)
    onehot = (jax.lax.broadcasted_iota(jnp.int32, (K, hb * w), 0)
              == lab2).astype(jnp.float32)
    c_sel = jax.lax.dot_general(
        centers, onehot, (((1,), (0,)), ((), ())),
        preferred_element_type=jnp.float32)
    diff = x - c_sel
    norm2 = jnp.sum(diff * diff, axis=0, keepdims=True)
    norm = jnp.sqrt(norm2)
    h = jnp.maximum(norm - DELTA_VAR, 0.0)
    var_b = jnp.sum(h * h)

    @pl.when(i == 0)
    def _():
        out_ref[0, 0] = 0.0

    out_ref[0, 0] += var_b / K

    @pl.when(i == NBLK - 1)
    def _():
        delta_reg = float(np.sqrt(centers.shape[0]))
        n2 = jnp.sum(centers * centers, axis=0)
        gram = jax.lax.dot_general(
            centers, centers, (((0,), (0,)), ((), ())),
            preferred_element_type=jnp.float32)
        sq = jnp.maximum(n2[:, None] + n2[None, :] - 2.0 * gram, 0.0)
        eye = jnp.eye(K, dtype=jnp.float32)
        cnorm = jnp.sqrt(sq + eye)
        hinge = (jnp.maximum(2.0 * DELTA_DIST - cnorm, 0.0) ** 2) * (1.0 - eye)
        dist_term = jnp.sum(hinge) / (K * (K - 1))
        reg_term = jnp.sum(jnp.maximum(jnp.sqrt(n2) - delta_reg, 0.0)) / K
        out_ref[0, 0] += dist_term + reg_term


def kernel(data, labels, cluster_ids):
    D, H, W = data.shape
    K = cluster_ids.shape[0]
    NC, NS = 2, 16
    NW = NC * NS
    HPB = 1

    mesh = plsc.VectorSubcoreMesh(core_axis_name="c", subcore_axis_name="s")
    sc_fn = pl.kernel(
        functools.partial(_sc_segsum, D, H, W, K, NC, NW, HPB),
        mesh=mesh,
        compiler_params=pltpu.CompilerParams(needs_layout_passes=False),
        out_type=[
            jax.ShapeDtypeStruct((NW, D, K), jnp.float32),
            jax.ShapeDtypeStruct((NW, K), jnp.float32),
        ],
        scratch_types=[
            pltpu.VMEM((D, HPB, W), jnp.float32),
            pltpu.VMEM((D, HPB, W), jnp.float32),
            pltpu.VMEM((HPB, W), jnp.int32),
            pltpu.VMEM((HPB, W), jnp.int32),
            pltpu.VMEM((K * D * 16 + K * 16,), jnp.float32),
            pltpu.VMEM((D, K), jnp.float32),
            pltpu.VMEM((K,), jnp.float32),
            pltpu.SemaphoreType.DMA,
            pltpu.SemaphoreType.DMA,
            pltpu.SemaphoreType.DMA,
            pltpu.SemaphoreType.DMA,
        ],
    )
    sums_pp, counts_pp = sc_fn(data, labels)

    HB = 16
    NBLK = H // HB
    out = pl.pallas_call(
        functools.partial(_phase2_body, K, NW, NBLK),
        grid=(NBLK,),
        in_specs=[
            pl.BlockSpec((D, HB, W), lambda i: (0, i, 0)),
            pl.BlockSpec((HB, W), lambda i: (i, 0)),
            pl.BlockSpec((NW, D, K), lambda i: (0, 0, 0)),
            pl.BlockSpec((NW, K), lambda i: (0, 0)),
        ],
        out_specs=pl.BlockSpec(memory_space=pltpu.SMEM),
        out_shape=jax.ShapeDtypeStruct((1, 1), jnp.float32),
    )(data, labels, sums_pp, counts_pp)

    return out[0, 0]


# speedup vs baseline: 2.4438x; 1.3319x over previous
import functools

import jax
import jax.numpy as jnp
import numpy as np
from jax import lax
from jax.experimental import pallas as pl
from jax.experimental.pallas import tpu as pltpu
from jax.experimental.pallas import tpu_sc as plsc

DELTA_VAR = 1.0
DELTA_DIST = 2.0


def _sc_segsum(D, HSC, W, K, NC, NW, HPB, data_hbm, lab_hbm, sums_pp,
               counts_pp, dbuf0, dbuf1, lbuf0, lbuf1, acc, acc2, cred,
               sem0, sem1, lsem0, lsem1):
    HPW = HSC // NW
    NBLK = HPW // HPB

    wid = lax.axis_index("s") * NC + lax.axis_index("c")
    lane = lax.iota(jnp.int32, 16)
    ones = jnp.full((16,), 1.0, jnp.float32)
    zeros = jnp.zeros((16,), jnp.float32)

    @plsc.parallel_loop(0, (K * D * 16 + K * 16) // 16)
    def zero_body(j):
        acc[pl.ds(j * 16, 16)] = zeros

    dbufs = (dbuf0, dbuf1)
    lbufs = (lbuf0, lbuf1)
    sems = (sem0, sem1)
    lsems = (lsem0, lsem1)
    h_base = wid * HPW

    def start(blk):
        b = blk % 2
        pltpu.async_copy(data_hbm.at[:, pl.ds(h_base + blk * HPB, HPB), :],
                         dbufs[b], sems[b])
        pltpu.async_copy(lab_hbm.at[pl.ds(h_base + blk * HPB, HPB), :],
                         lbufs[b], lsems[b])

    def wait(blk):
        b = blk % 2
        pltpu.make_async_copy(data_hbm.at[:, pl.ds(h_base + blk * HPB, HPB), :],
                              dbufs[b], sems[b]).wait()
        pltpu.make_async_copy(lab_hbm.at[pl.ds(h_base + blk * HPB, HPB), :],
                              lbufs[b], lsems[b]).wait()

    start(0)
    for blk in range(NBLK):
        b = blk % 2
        wait(blk)
        if blk + 1 < NBLK:
            start(blk + 1)
        dbuf = dbufs[b]
        lbuf = lbufs[b]
        for row in range(HPB):
            @plsc.parallel_loop(0, W // 16, unroll=2)
            def g_body(g):
                col = g * 16
                lab16 = lbuf[row, pl.ds(col, 16)]
                base = lab16 * 16 + lane
                plsc.addupdate_scatter(acc, [K * D * 16 + base], ones)
                for d in range(D):
                    v = dbuf[d, row, pl.ds(col, 16)]
                    plsc.addupdate_scatter(acc, [base + d * (K * 16)], v)

    @plsc.parallel_loop(0, D)
    def red_body(dd):
        res = zeros
        for labk in range(K):
            row16 = acc[pl.ds(dd * (K * 16) + labk * 16, 16)]
            s = jnp.sum(row16)
            res = jnp.where(lane == labk, s, res)
        acc2[dd, :] = res

    cres = zeros
    for labk in range(K):
        s = jnp.sum(acc[pl.ds(K * D * 16 + labk * 16, 16)])
        cres = jnp.where(lane == labk, s, cres)
    cred[...] = cres

    pltpu.sync_copy(acc2, sums_pp.at[wid])
    pltpu.sync_copy(cred, counts_pp.at[wid])


def _phase1_body(K, NBLK, data_ref, lab_ref, sums_ref, counts_ref):
    i = pl.program_id(0)
    hb, w = lab_ref.shape
    D = sums_ref.shape[0]
    x = data_ref[...].reshape(D, hb * w)
    lab2 = lab_ref[...].reshape(1, hb * w)
    onehot = (jax.lax.broadcasted_iota(jnp.int32, (K, hb * w), 0)
              == lab2).astype(jnp.float32)
    bsums = jax.lax.dot_general(
        x, onehot, (((1,), (1,)), ((), ())),
        preferred_element_type=jnp.float32)
    bcounts = jnp.sum(onehot, axis=1, keepdims=True).T

    @pl.when(i == 0)
    def _():
        sums_ref[...] = jnp.zeros_like(sums_ref)
        counts_ref[...] = jnp.zeros_like(counts_ref)

    sums_ref[...] += bsums
    counts_ref[...] += bcounts


def _phase2_body(K, NW, NBLK, data_ref, lab_ref, sums_ref, counts_ref,
                 sums_tc_ref, counts_tc_ref, out_ref):
    i = pl.program_id(0)
    hb, w = lab_ref.shape
    D = sums_ref.shape[1]
    sums = jnp.sum(sums_ref[...], axis=0) + sums_tc_ref[...]
    counts = (jnp.sum(counts_ref[...], axis=0, keepdims=True)
              + counts_tc_ref[...])
    centers = sums / counts
    x = data_ref[...].reshape(D, hb * w)
    lab2 = lab_ref[...].reshape(1, hb * w)
    onehot = (jax.lax.broadcasted_iota(jnp.int32, (K, hb * w), 0)
              == lab2).astype(jnp.float32)
    c_sel = jax.lax.dot_general(
        centers, onehot, (((1,), (0,)), ((), ())),
        preferred_element_type=jnp.float32)
    diff = x - c_sel
    norm2 = jnp.sum(diff * diff, axis=0, keepdims=True)
    norm = jnp.sqrt(norm2)
    h = jnp.maximum(norm - DELTA_VAR, 0.0)
    var_b = jnp.sum(h * h)

    @pl.when(i == 0)
    def _():
        out_ref[0, 0] = 0.0

    out_ref[0, 0] += var_b / K

    @pl.when(i == NBLK - 1)
    def _():
        delta_reg = float(np.sqrt(centers.shape[0]))
        n2 = jnp.sum(centers * centers, axis=0)
        gram = jax.lax.dot_general(
            centers, centers, (((0,), (0,)), ((), ())),
            preferred_element_type=jnp.float32)
        sq = jnp.maximum(n2[:, None] + n2[None, :] - 2.0 * gram, 0.0)
        eye = jnp.eye(K, dtype=jnp.float32)
        cnorm = jnp.sqrt(sq + eye)
        hinge = (jnp.maximum(2.0 * DELTA_DIST - cnorm, 0.0) ** 2) * (1.0 - eye)
        dist_term = jnp.sum(hinge) / (K * (K - 1))
        reg_term = jnp.sum(jnp.maximum(jnp.sqrt(n2) - delta_reg, 0.0)) / K
        out_ref[0, 0] += dist_term + reg_term


def kernel(data, labels, cluster_ids):
    D, H, W = data.shape
    K = cluster_ids.shape[0]
    NC, NS = 2, 16
    NW = NC * NS
    HPB = 1
    HSC = 160

    mesh = plsc.VectorSubcoreMesh(core_axis_name="c", subcore_axis_name="s")
    sc_fn = pl.kernel(
        functools.partial(_sc_segsum, D, HSC, W, K, NC, NW, HPB),
        mesh=mesh,
        compiler_params=pltpu.CompilerParams(needs_layout_passes=False),
        out_type=[
            jax.ShapeDtypeStruct((NW, D, K), jnp.float32),
            jax.ShapeDtypeStruct((NW, K), jnp.float32),
        ],
        scratch_types=[
            pltpu.VMEM((D, HPB, W), jnp.float32),
            pltpu.VMEM((D, HPB, W), jnp.float32),
            pltpu.VMEM((HPB, W), jnp.int32),
            pltpu.VMEM((HPB, W), jnp.int32),
            pltpu.VMEM((K * D * 16 + K * 16,), jnp.float32),
            pltpu.VMEM((D, K), jnp.float32),
            pltpu.VMEM((K,), jnp.float32),
            pltpu.SemaphoreType.DMA,
            pltpu.SemaphoreType.DMA,
            pltpu.SemaphoreType.DMA,
            pltpu.SemaphoreType.DMA,
        ],
    )
    sums_pp, counts_pp = sc_fn(data, labels)

    HB = 16
    OFF = HSC // HB
    NBLK_TC1 = (H - HSC) // HB
    sums_tc, counts_tc = pl.pallas_call(
        functools.partial(_phase1_body, K, NBLK_TC1),
        grid=(NBLK_TC1,),
        in_specs=[
            pl.BlockSpec((D, HB, W), lambda i: (0, i + OFF, 0)),
            pl.BlockSpec((HB, W), lambda i: (i + OFF, 0)),
        ],
        out_specs=[
            pl.BlockSpec((D, K), lambda i: (0, 0)),
            pl.BlockSpec((1, K), lambda i: (0, 0)),
        ],
        out_shape=[
            jax.ShapeDtypeStruct((D, K), jnp.float32),
            jax.ShapeDtypeStruct((1, K), jnp.float32),
        ],
    )(data, labels)

    NBLK = H // HB
    out = pl.pallas_call(
        functools.partial(_phase2_body, K, NW, NBLK),
        grid=(NBLK,),
        in_specs=[
            pl.BlockSpec((D, HB, W), lambda i: (0, i, 0)),
            pl.BlockSpec((HB, W), lambda i: (i, 0)),
            pl.BlockSpec((NW, D, K), lambda i: (0, 0, 0)),
            pl.BlockSpec((NW, K), lambda i: (0, 0)),
            pl.BlockSpec((D, K), lambda i: (0, 0)),
            pl.BlockSpec((1, K), lambda i: (0, 0)),
        ],
        out_specs=pl.BlockSpec(memory_space=pltpu.SMEM),
        out_shape=jax.ShapeDtypeStruct((1, 1), jnp.float32),
    )(data, labels, sums_pp, counts_pp, sums_tc, counts_tc)

    return out[0, 0]
